# Initial kernel scaffold; baseline (speedup 1.0000x reference)
import jax
import jax.numpy as jnp
from jax.experimental import pallas as pl


def kernel(x, edge_index, edge_weight, W_ih0, W_hh0, b_ih0, b_hh0, W_ih1, W_hh1, b_ih1, b_hh1, Wg0, bg0, Wg1, bg1, ln0_g, ln0_b, ln1_g, ln1_b, Wp0, bp0, Wp1, bp1):
    raise NotImplementedError("write your pallas kernel here")


# speedup vs baseline: 2.2346x; 2.2346x over previous
import functools

import jax
import jax.numpy as jnp
from jax import lax
from jax.experimental import pallas as pl
from jax.experimental.pallas import tpu as pltpu
from jax.experimental.pallas import tpu_sc as plsc

N = 50000
E = 800000
F = 16
T = 8
H = 64
G4 = 4 * H

NC = 2
NS = 16
NW = NC * NS
EPW = E // NW
CH = 1000
NCH = EPW // CH
NGRP = CH // 16 + 1
CHB = CH + 8

N2 = N // 2
RPT = 1568
ACC = NS * RPT
NOUT = 50176
DRPT = 3136
DACC = NS * DRPT
BR = 64
CB = 1024
PRPT = 12544
TRPT = PRPT // NS
EPW2 = E // NS
NCH2 = EPW2 // CH

BNL = 512
GRIDL = (N + BNL - 1) // BNL
BN = 2048
GRIDE = (N + BN - 1) // BN

_mesh = functools.partial(
    plsc.VectorSubcoreMesh, core_axis_name="c", subcore_axis_name="s",
    num_cores=NC, num_subcores=NS)


def _dot(a, b):
  return lax.dot_general(
      a, b, (((1,), (1,)), ((), ())),
      preferred_element_type=jnp.float32,
      precision=lax.Precision.HIGHEST)


def _lstm_body(x_ref, wih0_ref, whh0_ref, b0_ref, wih1_ref, whh1_ref, b1_ref,
               out_ref):
  xb = x_ref[...]
  xg0 = (_dot(xb, wih0_ref[...]) + b0_ref[...]).reshape(BNL, T, G4)
  h = jnp.zeros((BNL, H), jnp.float32)
  c = jnp.zeros((BNL, H), jnp.float32)
  ys = []
  for t in range(T):
    g = xg0[:, t, :] + _dot(h, whh0_ref[...])
    ii, ff, gg, oo = (g[:, :H], g[:, H:2*H], g[:, 2*H:3*H], g[:, 3*H:])
    c = jax.nn.sigmoid(ff) * c + jax.nn.sigmoid(ii) * jnp.tanh(gg)
    h = jax.nn.sigmoid(oo) * jnp.tanh(c)
    ys.append(h)
  y0 = jnp.stack(ys, axis=1).reshape(BNL * T, H)
  xg1 = (_dot(y0, wih1_ref[...]) + b1_ref[...]).reshape(BNL, T, G4)
  h = jnp.zeros((BNL, H), jnp.float32)
  c = jnp.zeros((BNL, H), jnp.float32)
  for t in range(T):
    g = xg1[:, t, :] + _dot(h, whh1_ref[...])
    ii, ff, gg, oo = (g[:, :H], g[:, H:2*H], g[:, 2*H:3*H], g[:, 3*H:])
    c = jax.nn.sigmoid(ff) * c + jax.nn.sigmoid(ii) * jnp.tanh(gg)
    h = jax.nn.sigmoid(oo) * jnp.tanh(c)
  out_ref[...] = h


def _lstm_tc(x2d, wih0, whh0, b0, wih1, whh1, b1):
  full = lambda shape: pl.BlockSpec(shape, lambda i: (0,) * len(shape))
  return pl.pallas_call(
      _lstm_body,
      grid=(GRIDL,),
      in_specs=[
          pl.BlockSpec((BNL * T, F), lambda i: (i, 0)),
          full((G4, F)), full((G4, H)), full((1, G4)),
          full((G4, H)), full((G4, H)), full((1, G4)),
      ],
      out_specs=pl.BlockSpec((BNL, H), lambda i: (i, 0)),
      out_shape=jax.ShapeDtypeStruct((N, H), jnp.float32),
  )(x2d, wih0, whh0, b0, wih1, whh1, b1)


def _deg_body(c_hbm, w_hbm, out_hbm, acc_sh, cstage, wstage, wtail, zbuf,
              ssem):
  cid = lax.axis_index("c")
  sid = lax.axis_index("s")
  wid = cid * NS + sid
  ebase = wid * EPW
  iota = lax.iota(jnp.int32, 16)

  def zi(k, c):
    zbuf[pl.ds(k * 16, 16)] = jnp.zeros((16,), jnp.float32)
    return c
  lax.fori_loop(0, DRPT // 16, zi, 0)
  pltpu.sync_copy(zbuf, acc_sh.at[pl.ds(sid * DRPT, DRPT)])
  plsc.subcore_barrier()

  def chunk(ch, carry):
    off = ebase + ch * CH
    pltpu.sync_copy(c_hbm.at[pl.ds(off, CH)], cstage.at[pl.ds(0, CH)])
    pltpu.sync_copy(w_hbm.at[pl.ds(off, CH)], wstage.at[pl.ds(0, CH)])

    def grp(k, carry2):
      cv = cstage[pl.ds(k * 16, 16)]
      pltpu.async_copy(
          wstage.at[pl.ds(k * 16, 16)], acc_sh.at[cv], ssem, add=True).wait()
      return carry2

    carry = lax.fori_loop(0, NGRP - 1, grp, carry)
    k = NGRP - 1
    lanemask = iota < (CH - (NGRP - 1) * 16)
    cv = cstage[pl.ds(k * 16, 16)]
    wv = wstage[pl.ds(k * 16, 16)]
    cv = jnp.where(lanemask, cv, N + wid)
    wtail[...] = jnp.where(lanemask, wv, 0.0)
    pltpu.async_copy(wtail, acc_sh.at[cv], ssem, add=True).wait()
    return carry

  lax.fori_loop(0, NCH, chunk, 0)
  plsc.subcore_barrier()
  pltpu.sync_copy(acc_sh.at[pl.ds(sid * DRPT, DRPT)], zbuf)
  pltpu.sync_copy(zbuf, out_hbm.at[pl.ds(cid * DACC + sid * DRPT, DRPT)])


def _deg_sc(col, ew):
  k = pl.kernel(
      _deg_body,
      out_type=jax.ShapeDtypeStruct((NC * DACC,), jnp.float32),
      mesh=_mesh(),
      scratch_types=[
          pltpu.VMEM_SHARED((DACC,), jnp.float32),
          pltpu.VMEM((CHB,), jnp.int32),
          pltpu.VMEM((CHB,), jnp.float32),
          pltpu.VMEM((16,), jnp.float32),
          pltpu.VMEM((DRPT,), jnp.float32),
          pltpu.SemaphoreType.DMA,
      ])
  return k(col, ew)


def _msg_body(r_hbm, c_hbm, w_hbm, hs_hbm, out_hbm, acc_sh,
              ridx, cbidx, wcmp, rows128, zrows, gsem, ssem):
  cid = lax.axis_index("c")
  sid = lax.axis_index("s")
  ebase = sid * EPW2
  iota = lax.iota(jnp.int32, 16)
  ones_i = jnp.ones((16,), jnp.int32)
  z16 = jnp.zeros((16,), jnp.float32)

  for i in range(BR):
    for q in range(8):
      zrows[i, pl.ds(q * 16, 16)] = z16

  for p in range(2):
    lo = cid * (2 * PRPT) + p * PRPT
    abase = sid * TRPT
    for j in range(12):
      pltpu.sync_copy(zrows, acc_sh.at[pl.ds(abase + j * BR, BR)])
    pltpu.sync_copy(zrows.at[pl.ds(0, 16)],
                    acc_sh.at[pl.ds(abase + 12 * BR, 16)])
    plsc.subcore_barrier()

    def chunk(ch, c0):
      off = ebase + ch * CH
      pltpu.sync_copy(r_hbm.at[pl.ds(off, CH)], ridx.at[pl.ds(0, CH)])
      pltpu.sync_copy(c_hbm.at[pl.ds(off, CH)], cbidx.at[pl.ds(0, CH)])
      pltpu.sync_copy(w_hbm.at[pl.ds(off, CH)], wcmp.at[pl.ds(0, CH)])

      def xform(k, c):
        o = k * 16
        lanemask = (o + iota) < CH
        rv = ridx[pl.ds(o, 16)]
        cv = cbidx[pl.ds(o, 16)]
        wv = wcmp[pl.ds(o, 16)]
        m = (cv >= lo) & (cv < lo + PRPT) & lanemask
        ridx[pl.ds(o, 16)] = jnp.clip(rv, 0, N - 1)
        cbidx[pl.ds(o, 16)] = jnp.where(m, cv - lo, 0)
        wcmp[pl.ds(o, 16)] = jnp.where(m, wv, 0.0)
        return c

      lax.fori_loop(0, CB // 16, xform, 0)

      def batch(b, carry):
        pltpu.async_copy(
            hs_hbm.at[ridx.at[pl.ds(b * BR, BR)]], rows128, gsem).wait()
        for g in range(BR // 16):
          wv = wcmp[pl.ds(b * BR + g * 16, 16)]
          for j in range(16):
            wb = wv.at[ones_i * j].get(mode="promise_in_bounds")
            i = g * 16 + j
            for q in range(4):
              rows128[i, pl.ds(q * 16, 16)] = (
                  rows128[i, pl.ds(q * 16, 16)] * wb)
        descs = []
        for g in range(BR // 16):
          cbv = cbidx[pl.ds(b * BR + g * 16, 16)]
          descs.append(pltpu.async_copy(
              rows128.at[pl.ds(g * 16, 16)], acc_sh.at[cbv], ssem,
              add=True))
        for d in descs:
          d.wait()
        return carry

      lax.fori_loop(0, CB // BR, batch, 0)
      return c0

    lax.fori_loop(0, NCH2, chunk, 0)
    plsc.subcore_barrier()

    for j in range(12):
      pltpu.sync_copy(acc_sh.at[pl.ds(abase + j * BR, BR)], zrows)
      pltpu.sync_copy(zrows, out_hbm.at[pl.ds(lo + abase + j * BR, BR)])
    pltpu.sync_copy(acc_sh.at[pl.ds(abase + 12 * BR, 16)],
                    zrows.at[pl.ds(0, 16)])
    pltpu.sync_copy(zrows.at[pl.ds(0, 16)],
                    out_hbm.at[pl.ds(lo + abase + 12 * BR, 16)])
    for i in range(BR):
      for q in range(8):
        zrows[i, pl.ds(q * 16, 16)] = z16
    plsc.subcore_barrier()


def _msg_sc(row, col, ew, hs):
  k = pl.kernel(
      _msg_body,
      out_type=jax.ShapeDtypeStruct((NOUT, 128), jnp.float32),
      mesh=_mesh(),
      scratch_types=[
          pltpu.VMEM_SHARED((PRPT, 128), jnp.float32),
          pltpu.VMEM((CB,), jnp.int32),
          pltpu.VMEM((CB,), jnp.int32),
          pltpu.VMEM((CB,), jnp.float32),
          pltpu.VMEM((BR, 128), jnp.float32),
          pltpu.VMEM((BR, 128), jnp.float32),
          pltpu.SemaphoreType.DMA,
          pltpu.SemaphoreType.DMA,
      ])
  return k(row, col, ew, hs)


def _prescale_body(tf_ref, d0_ref, d1_ref, wg0_ref, hs_ref, dinv_ref):
  deg = d0_ref[...] + d1_ref[...] + 1.0
  dinv = lax.rsqrt(jnp.maximum(deg, 1e-12))
  hs_ref[:, :H] = _dot(tf_ref[...], wg0_ref[...]) * dinv
  hs_ref[:, H:] = jnp.zeros((BN, 128 - H), jnp.float32)
  dinv_ref[...] = dinv


def _prescale_tc(tfeat, d0, d1, wg0):
  return pl.pallas_call(
      _prescale_body,
      grid=(GRIDE,),
      in_specs=[
          pl.BlockSpec((BN, H), lambda i: (i, 0)),
          pl.BlockSpec((BN, 1), lambda i: (i, 0)),
          pl.BlockSpec((BN, 1), lambda i: (i, 0)),
          pl.BlockSpec((H, H), lambda i: (0, 0)),
      ],
      out_specs=[
          pl.BlockSpec((BN, 128), lambda i: (i, 0)),
          pl.BlockSpec((BN, 1), lambda i: (i, 0)),
      ],
      out_shape=[
          jax.ShapeDtypeStruct((N, 128), jnp.float32),
          jax.ShapeDtypeStruct((N, 1), jnp.float32),
      ],
  )(tfeat, d0, d1, wg0)


def _ln(x, g, b):
  m = jnp.mean(x, axis=1, keepdims=True)
  xc = x - m
  v = jnp.mean(xc * xc, axis=1, keepdims=True)
  return xc * lax.rsqrt(v + 1e-5) * g + b


def _mid_body(s0_ref, hs0_ref, dinv_ref, wg1_ref, bg0_ref,
              ln0g_ref, ln0b_ref, hs1_ref):
  dinv = dinv_ref[...]
  pre = dinv * (s0_ref[:, :H] + hs0_ref[:, :H]) + bg0_ref[...]
  g0 = jnp.maximum(_ln(pre, ln0g_ref[...], ln0b_ref[...]), 0.0)
  hs1_ref[:, :H] = _dot(g0, wg1_ref[...]) * dinv
  hs1_ref[:, H:] = jnp.zeros((BN, 128 - H), jnp.float32)


def _mid_tc(s0, hs0, dinv, wg1, bg0, ln0g, ln0b):
  vec = lambda: pl.BlockSpec((1, H), lambda i: (0, 0))
  return pl.pallas_call(
      _mid_body,
      grid=(GRIDE,),
      in_specs=[
          pl.BlockSpec((BN, 128), lambda i: (i, 0)),
          pl.BlockSpec((BN, 128), lambda i: (i, 0)),
          pl.BlockSpec((BN, 1), lambda i: (i, 0)),
          pl.BlockSpec((H, H), lambda i: (0, 0)),
          vec(), vec(), vec(),
      ],
      out_specs=pl.BlockSpec((BN, 128), lambda i: (i, 0)),
      out_shape=jax.ShapeDtypeStruct((N, 128), jnp.float32),
  )(s0, hs0, dinv, wg1, bg0, ln0g, ln0b)


def _out_body(s1_ref, hs1_ref, dinv_ref, tf_ref, wp0a_ref,
              wp0b_ref, wp1_ref, bg1_ref, ln1g_ref, ln1b_ref, bp0_ref,
              y_ref):
  pre = dinv_ref[...] * (s1_ref[:, :H] + hs1_ref[:, :H]) + bg1_ref[...]
  g1 = _ln(pre, ln1g_ref[...], ln1b_ref[...])
  hid = jnp.maximum(
      _dot(tf_ref[...], wp0a_ref[...]) + _dot(g1, wp0b_ref[...])
      + bp0_ref[...], 0.0)
  y_ref[...] = _dot(hid, wp1_ref[...])


def _out_tc(s1, hs1, dinv, tfeat, wp0a, wp0b, wp1, bg1, ln1g, ln1b, bp0):
  HH = H // 2
  return pl.pallas_call(
      _out_body,
      grid=(GRIDE,),
      in_specs=[
          pl.BlockSpec((BN, 128), lambda i: (i, 0)),
          pl.BlockSpec((BN, 128), lambda i: (i, 0)),
          pl.BlockSpec((BN, 1), lambda i: (i, 0)),
          pl.BlockSpec((BN, H), lambda i: (i, 0)),
          pl.BlockSpec((HH, H), lambda i: (0, 0)),
          pl.BlockSpec((HH, H), lambda i: (0, 0)),
          pl.BlockSpec((1, HH), lambda i: (0, 0)),
          pl.BlockSpec((1, H), lambda i: (0, 0)),
          pl.BlockSpec((1, H), lambda i: (0, 0)),
          pl.BlockSpec((1, H), lambda i: (0, 0)),
          pl.BlockSpec((1, HH), lambda i: (0, 0)),
      ],
      out_specs=pl.BlockSpec((BN, 1), lambda i: (i, 0)),
      out_shape=jax.ShapeDtypeStruct((N, 1), jnp.float32),
  )(s1, hs1, dinv, tfeat, wp0a, wp0b, wp1, bg1, ln1g, ln1b, bp0)


def kernel(x, edge_index, edge_weight, W_ih0, W_hh0, b_ih0, b_hh0, W_ih1,
           W_hh1, b_ih1, b_hh1, Wg0, bg0, Wg1, bg1, ln0_g, ln0_b, ln1_g,
           ln1_b, Wp0, bp0, Wp1, bp1):
  x2d = jnp.transpose(x, (0, 2, 1)).reshape(N * T, F)
  b0 = (b_ih0 + b_hh0)[None, :]
  b1 = (b_ih1 + b_hh1)[None, :]
  row = edge_index[0]
  col = edge_index[1]

  degp = _deg_sc(col, edge_weight).reshape(NC, DACC)
  d0 = degp[0, :N, None]
  d1 = degp[1, :N, None]

  tfeat = _lstm_tc(x2d, W_ih0, W_hh0, b0, W_ih1, W_hh1, b1)

  hs0, dinv = _prescale_tc(tfeat, d0, d1, Wg0)

  s0 = _msg_sc(row, col, edge_weight, hs0)

  hs1 = _mid_tc(s0, hs0, dinv, Wg1, bg0[None, :], ln0_g[None, :],
                ln0_b[None, :])

  s1 = _msg_sc(row, col, edge_weight, hs1)

  y = _out_tc(s1, hs1, dinv, tfeat, Wp0[:, :H], Wp0[:, H:], Wp1,
              bg1[None, :], ln1_g[None, :], ln1_b[None, :], bp0[None, :])
  return y + bp1[None, :]
